# baseline (speedup 1.0000x reference)
import functools

import jax
import jax.numpy as jnp
from jax import lax
from jax.experimental import pallas as pl
from jax.experimental.pallas import tpu as pltpu
from jax.experimental.pallas import tpu_sc as plsc

NC = 2
NS = 16
NW = NC * NS
CHUNK = 112


def _tc_matmul(h, w_all):
    n, d = h.shape
    rp, _, hd = w_all.shape
    bn = 1000

    def body(h_ref, w_ref, o_ref):
        o_ref[...] = jnp.dot(h_ref[...], w_ref[0],
                             preferred_element_type=jnp.float32)

    return pl.pallas_call(
        body,
        grid=(rp, n // bn),
        in_specs=[
            pl.BlockSpec((bn, d), lambda r, i: (i, 0)),
            pl.BlockSpec((1, d, hd), lambda r, i: (r, 0, 0)),
        ],
        out_specs=pl.BlockSpec((bn, hd), lambda r, i: (r * (n // bn) + i, 0)),
        out_shape=jax.ShapeDtypeStruct((rp * n, hd), jnp.float32),
    )(h, w_all)


def _tc_matmul_fused(parts, hw_prev, w_all, n):
    _, _, hd = parts.shape
    rp, d, _ = w_all.shape
    bn = 1000
    off = (hw_prev.shape[0] - n) // bn

    def body(p_ref, s_ref, w_ref, o_ref):
        h = jnp.maximum(p_ref[0] + p_ref[1] + s_ref[...], 0.0)
        o_ref[...] = jnp.dot(h, w_ref[0], preferred_element_type=jnp.float32)

    return pl.pallas_call(
        body,
        grid=(rp, n // bn),
        in_specs=[
            pl.BlockSpec((2, bn, hd), lambda r, i: (0, i, 0)),
            pl.BlockSpec((bn, hd), lambda r, i: (off + i, 0)),
            pl.BlockSpec((1, d, hd), lambda r, i: (r, 0, 0)),
        ],
        out_specs=pl.BlockSpec((bn, hd), lambda r, i: (r * (n // bn) + i, 0)),
        out_shape=jax.ShapeDtypeStruct((rp * n, hd), jnp.float32),
    )(parts, hw_prev, w_all)


def _tc_combine(parts, hw_full, n, relu):
    _, _, hd = parts.shape
    rn = hw_full.shape[0] - n
    bn = 1000
    off = rn // bn

    def body(p_ref, s_ref, o_ref):
        v = p_ref[0] + p_ref[1] + s_ref[...]
        o_ref[...] = jnp.maximum(v, 0.0) if relu else v

    return pl.pallas_call(
        body,
        grid=(n // bn,),
        in_specs=[
            pl.BlockSpec((2, bn, hd), lambda i: (0, i, 0)),
            pl.BlockSpec((bn, hd), lambda i: (off + i, 0)),
        ],
        out_specs=pl.BlockSpec((bn, hd), lambda i: (i, 0)),
        out_shape=jax.ShapeDtypeStruct((n, hd), jnp.float32),
    )(parts, hw_full)


def _fence():
    plsc.subcore_barrier()
    pl.delay(3000)
    plsc.subcore_barrier()


def _sc_counts(gidx2, zeros_hbm_arr, ones_hbm_arr, rnp, s_steps):
    mesh = plsc.VectorSubcoreMesh(core_axis_name="c", subcore_axis_name="s")
    per_tile = rnp // NS

    @functools.partial(
        pl.kernel,
        out_type=jax.ShapeDtypeStruct((NC, rnp), jnp.float32),
        mesh=mesh,
        scratch_types=[
            pltpu.VMEM_SHARED((rnp,), jnp.float32),
            pltpu.VMEM((2, CHUNK), jnp.int32),
            pltpu.VMEM((CHUNK,), jnp.float32),
            pltpu.VMEM((per_tile,), jnp.float32),
        ] + [pltpu.SemaphoreType.DMA] * 4,
    )
    def k(g2_hbm, z_hbm, ones_hbm, out_hbm, acc_sh, idx_v, ones_v, buf_v,
          semi0, semi1, sems0, sems1):
        cid = lax.axis_index("c")
        sid = lax.axis_index("s")
        wid = sid * NC + cid
        semi = (semi0, semi1)
        sems_ = (sems0, sems1)

        pltpu.sync_copy(ones_hbm, ones_v)
        pltpu.sync_copy(z_hbm.at[pl.ds(0, per_tile)], buf_v)
        pltpu.sync_copy(buf_v, acc_sh.at[pl.ds(sid * per_tile, per_tile)])
        _fence()

        base = wid * s_steps

        def start_idx(b, c):
            pltpu.async_copy(g2_hbm.at[pl.ds((base + c) * CHUNK, CHUNK)],
                             idx_v.at[b], semi[b])

        def wait_idx(b):
            pltpu.make_async_copy(g2_hbm.at[pl.ds(0, CHUNK)], idx_v.at[b],
                                  semi[b]).wait()

        def start_scatter(b):
            pltpu.async_copy(ones_v, acc_sh.at[idx_v.at[b]], sems_[b],
                             add=True)

        def wait_scatter(b):
            pltpu.make_async_copy(ones_v, acc_sh.at[idx_v.at[b]],
                                  sems_[b]).wait()

        def body(b, c):
            wait_idx(b)
            start_scatter(b)
            wait_scatter(b)
            if isinstance(c, int):
                if c + 2 < s_steps:
                    start_idx(b, c + 2)
            else:
                @pl.when(c + 2 < s_steps)
                def _():
                    start_idx(b, c + 2)

        start_idx(0, 0)
        start_idx(1, 1)
        body(0, 0)
        body(1, 1)

        @pl.loop(1, s_steps // 2)
        def _(kk):
            c0 = 2 * kk
            for b in range(2):
                body(b, c0 + b)

        for c in range(2 * (s_steps // 2), s_steps):
            body(c % 2, c)

        _fence()
        pltpu.sync_copy(acc_sh.at[pl.ds(sid * per_tile, per_tile)], buf_v)
        pltpu.sync_copy(buf_v, out_hbm.at[cid, pl.ds(sid * per_tile, per_tile)])

    return k(gidx2, zeros_hbm_arr, ones_hbm_arr)


def _tc_norm(counts_parts, rn, rnp):
    rows = rnp // 128
    live = rn // 128

    def body(c_ref, o_ref):
        c = c_ref[0] + c_ref[1]
        row = lax.broadcasted_iota(jnp.int32, (rows, 128), 0)
        o_ref[...] = jnp.where((row < live) & (c > 0.0), 1.0 / c, 0.0)

    out = pl.pallas_call(
        body,
        grid=(1,),
        in_specs=[pl.BlockSpec((NC, rows, 128), lambda i: (0, 0, 0))],
        out_specs=pl.BlockSpec((rows, 128), lambda i: (0, 0)),
        out_shape=jax.ShapeDtypeStruct((rows, 128), jnp.float32),
    )(counts_parts.reshape(NC, rows, 128))
    return out.reshape(rnp)


def _sc_message(hw, edata, ntab, zeros_hbm_arr, n, hd, s_steps):
    mesh = plsc.VectorSubcoreMesh(core_axis_name="c", subcore_axis_name="s")
    nfull = n // CHUNK
    ntail = n - nfull * CHUNK
    nzch = nfull + (1 if ntail else 0)

    @functools.partial(
        pl.kernel,
        out_type=jax.ShapeDtypeStruct((NC, n, hd), jnp.float32),
        mesh=mesh,
        scratch_types=[
            pltpu.VMEM_SHARED((n, hd), jnp.float32),
            pltpu.VMEM((3, 3, CHUNK), jnp.int32),
            pltpu.VMEM((3, CHUNK), jnp.float32),
            pltpu.VMEM((3, CHUNK, hd), jnp.float32),
        ] + [pltpu.SemaphoreType.DMA] * 12,
    )
    def k(hw_hbm, ed_hbm, ntab_hbm, z_hbm, out_hbm,
          acc_sh, idx_v, nrm_v, rows_v, *sems):
        cid = lax.axis_index("c")
        sid = lax.axis_index("s")
        wid = sid * NC + cid
        semi = sems[0:3]
        semr = sems[3:6]
        semn = sems[6:9]
        sems_ = sems[9:12]

        pltpu.sync_copy(z_hbm, rows_v.at[0])
        for kk in range(-(-nzch // NS)):
            c = sid + kk * NS

            @pl.when(c < nfull)
            def _():
                pltpu.sync_copy(rows_v.at[0], acc_sh.at[pl.ds(c * CHUNK, CHUNK)])

            if ntail:
                @pl.when(c == nfull)
                def _():
                    pltpu.sync_copy(rows_v.at[0, pl.ds(0, ntail)],
                                    acc_sh.at[pl.ds(nfull * CHUNK, ntail)])

        _fence()

        base = wid * s_steps

        def start_idx(b, c):
            pltpu.async_copy(ed_hbm.at[base + c], idx_v.at[b], semi[b])

        def wait_idx(b):
            pltpu.make_async_copy(ed_hbm.at[base], idx_v.at[b], semi[b]).wait()

        def start_gathers(b):
            pltpu.async_copy(hw_hbm.at[idx_v.at[b, 0]], rows_v.at[b], semr[b])
            pltpu.async_copy(ntab_hbm.at[idx_v.at[b, 2]], nrm_v.at[b], semn[b])

        def wait_gathers(b):
            pltpu.make_async_copy(
                hw_hbm.at[idx_v.at[b, 0]], rows_v.at[b], semr[b]).wait()
            pltpu.make_async_copy(
                ntab_hbm.at[idx_v.at[b, 2]], nrm_v.at[b], semn[b]).wait()

        def scale(b):
            @pl.loop(0, CHUNK, step=16)
            def _(j):
                nv = nrm_v[b, pl.ds(j, 16)]
                for jj in range(16):
                    sv = nv[jj]
                    for kk in range(hd // 16):
                        sl = (b, j + jj, pl.ds(kk * 16, 16))
                        rows_v.at[*sl][...] = rows_v.at[*sl][...] * sv

        def start_scatter(b):
            pltpu.async_copy(rows_v.at[b], acc_sh.at[idx_v.at[b, 1]], sems_[b],
                             add=True)

        def wait_scatter(b):
            pltpu.make_async_copy(
                rows_v.at[b], acc_sh.at[idx_v.at[b, 1]], sems_[b]).wait()

        def body(i, c, first=False):
            wait_gathers(i)
            scale(i)
            start_scatter(i)
            s2 = (i + 2) % 3
            if first:
                start_idx(s2, c + 2)
                wait_idx(s2)
                start_gathers(s2)
            elif isinstance(c, int):
                wait_scatter(s2)
                if c + 2 < s_steps:
                    start_idx(s2, c + 2)
                    wait_idx(s2)
                    start_gathers(s2)
            else:
                wait_scatter(s2)

                @pl.when(c + 2 < s_steps)
                def _():
                    start_idx(s2, c + 2)
                    wait_idx(s2)
                    start_gathers(s2)

        start_idx(0, 0)
        wait_idx(0)
        start_gathers(0)
        start_idx(1, 1)
        wait_idx(1)
        start_gathers(1)

        body(0, 0, first=True)
        for c in range(1, 3):
            body(c % 3, c)

        @pl.loop(1, s_steps // 3)
        def _(kk):
            c0 = 3 * kk
            for i in range(3):
                body(i, c0 + i)

        for c in range(3 * (s_steps // 3), s_steps):
            body(c % 3, c)
        wait_scatter((s_steps - 1) % 3)

        _fence()
        for kk in range(-(-nzch // NS)):
            c = sid + kk * NS

            @pl.when(c < nfull)
            def _():
                pltpu.sync_copy(acc_sh.at[pl.ds(c * CHUNK, CHUNK)], rows_v.at[0])
                pltpu.sync_copy(rows_v.at[0],
                                out_hbm.at[cid, pl.ds(c * CHUNK, CHUNK)])

            if ntail:
                @pl.when(c == nfull)
                def _():
                    pltpu.sync_copy(acc_sh.at[pl.ds(nfull * CHUNK, ntail)],
                                    rows_v.at[0, pl.ds(0, ntail)])
                    pltpu.sync_copy(rows_v.at[0, pl.ds(0, ntail)],
                                    out_hbm.at[cid, pl.ds(nfull * CHUNK, ntail)])

    return k(hw, edata, ntab, zeros_hbm_arr)


def kernel(x, edge_index, edge_type, W1, W1_self, W2, W2_self):
    n, d = x.shape
    r = W1.shape[0]
    hd = W1.shape[2]
    e = edge_index.shape[1]
    src = edge_index[0]
    dst = edge_index[1]
    et = edge_type.astype(jnp.int32)

    rn = r * n
    rnp = rn + (-rn % 2048) + 2048

    gidx = et * n + src
    gidx2 = et * n + dst

    s_steps = -(-e // (NW * CHUNK))
    e_pad = NW * CHUNK * s_steps
    pad = e_pad - e
    pad_i = jnp.arange(pad, dtype=jnp.int32)
    gidx_p = jnp.concatenate([gidx, pad_i % rn])
    dst_p = jnp.concatenate([dst, pad_i % n])
    gidx2_p = jnp.concatenate([gidx2, rn + pad_i % 1024])
    ctot = e_pad // CHUNK
    edata = jnp.stack(
        [gidx_p.reshape(ctot, CHUNK), dst_p.reshape(ctot, CHUNK),
         gidx2_p.reshape(ctot, CHUNK)], axis=1)

    w1a = jnp.concatenate([W1, W1_self[None]], axis=0)
    w2a = jnp.concatenate([W2, W2_self[None]], axis=0)

    z1d = jnp.zeros((rnp // NS,), jnp.float32)
    z2d = jnp.zeros((CHUNK, hd), jnp.float32)
    ones_c = jnp.ones((CHUNK,), jnp.float32)

    counts_parts = _sc_counts(gidx2_p, z1d, ones_c, rnp, s_steps)
    ntab = _tc_norm(counts_parts, rn, rnp)

    hw1 = _tc_matmul(x, w1a)
    parts1 = _sc_message(hw1, edata, ntab, z2d, n, hd, s_steps)
    hw2 = _tc_matmul_fused(parts1, hw1, w2a, n)
    parts2 = _sc_message(hw2, edata, ntab, z2d, n, hd, s_steps)
    return _tc_combine(parts2, hw2, n, relu=False)


# speedup vs baseline: 1.0688x; 1.0688x over previous
import functools

import jax
import jax.numpy as jnp
from jax import lax
from jax.experimental import pallas as pl
from jax.experimental.pallas import tpu as pltpu
from jax.experimental.pallas import tpu_sc as plsc

NC = 2
NS = 16
NW = NC * NS
CHUNK = 112


def _tc_matmul(h, w_all):
    n, d = h.shape
    rp, _, hd = w_all.shape
    bn = 1000

    def body(h_ref, w_ref, o_ref):
        o_ref[...] = jnp.dot(h_ref[...], w_ref[0],
                             preferred_element_type=jnp.float32)

    return pl.pallas_call(
        body,
        grid=(n // bn, rp),
        in_specs=[
            pl.BlockSpec((bn, d), lambda i, r: (i, 0)),
            pl.BlockSpec((1, d, hd), lambda i, r: (r, 0, 0)),
        ],
        out_specs=pl.BlockSpec((bn, hd), lambda i, r: (r * (n // bn) + i, 0)),
        out_shape=jax.ShapeDtypeStruct((rp * n, hd), jnp.float32),
    )(h, w_all)


def _tc_matmul_fused(parts, hw_prev, w_all, n):
    _, _, hd = parts.shape
    rp, d, _ = w_all.shape
    bn = 1000
    off = (hw_prev.shape[0] - n) // bn

    def body(p_ref, s_ref, w_ref, o_ref):
        h = jnp.maximum(p_ref[0] + p_ref[1] + s_ref[...], 0.0)
        o_ref[...] = jnp.dot(h, w_ref[0], preferred_element_type=jnp.float32)

    return pl.pallas_call(
        body,
        grid=(n // bn, rp),
        in_specs=[
            pl.BlockSpec((2, bn, hd), lambda i, r: (0, i, 0)),
            pl.BlockSpec((bn, hd), lambda i, r: (off + i, 0)),
            pl.BlockSpec((1, d, hd), lambda i, r: (r, 0, 0)),
        ],
        out_specs=pl.BlockSpec((bn, hd), lambda i, r: (r * (n // bn) + i, 0)),
        out_shape=jax.ShapeDtypeStruct((rp * n, hd), jnp.float32),
    )(parts, hw_prev, w_all)


def _tc_combine(parts, hw_full, n, relu):
    _, _, hd = parts.shape
    rn = hw_full.shape[0] - n
    bn = 1000
    off = rn // bn

    def body(p_ref, s_ref, o_ref):
        v = p_ref[0] + p_ref[1] + s_ref[...]
        o_ref[...] = jnp.maximum(v, 0.0) if relu else v

    return pl.pallas_call(
        body,
        grid=(n // bn,),
        in_specs=[
            pl.BlockSpec((2, bn, hd), lambda i: (0, i, 0)),
            pl.BlockSpec((bn, hd), lambda i: (off + i, 0)),
        ],
        out_specs=pl.BlockSpec((bn, hd), lambda i: (i, 0)),
        out_shape=jax.ShapeDtypeStruct((n, hd), jnp.float32),
    )(parts, hw_full)


def _fence():
    plsc.subcore_barrier()
    pl.delay(3000)
    plsc.subcore_barrier()


def _sc_counts(gidx2, zeros_hbm_arr, ones_hbm_arr, rnp, s_steps):
    mesh = plsc.VectorSubcoreMesh(core_axis_name="c", subcore_axis_name="s")
    per_tile = rnp // NS

    @functools.partial(
        pl.kernel,
        out_type=jax.ShapeDtypeStruct((NC, rnp), jnp.float32),
        mesh=mesh,
        scratch_types=[
            pltpu.VMEM_SHARED((rnp,), jnp.float32),
            pltpu.VMEM((2, CHUNK), jnp.int32),
            pltpu.VMEM((CHUNK,), jnp.float32),
            pltpu.VMEM((per_tile,), jnp.float32),
        ] + [pltpu.SemaphoreType.DMA] * 4,
    )
    def k(g2_hbm, z_hbm, ones_hbm, out_hbm, acc_sh, idx_v, ones_v, buf_v,
          semi0, semi1, sems0, sems1):
        cid = lax.axis_index("c")
        sid = lax.axis_index("s")
        wid = sid * NC + cid
        semi = (semi0, semi1)
        sems_ = (sems0, sems1)

        pltpu.sync_copy(ones_hbm, ones_v)
        pltpu.sync_copy(z_hbm.at[pl.ds(0, per_tile)], buf_v)
        pltpu.sync_copy(buf_v, acc_sh.at[pl.ds(sid * per_tile, per_tile)])
        _fence()

        base = wid * s_steps

        def start_idx(b, c):
            pltpu.async_copy(g2_hbm.at[pl.ds((base + c) * CHUNK, CHUNK)],
                             idx_v.at[b], semi[b])

        def wait_idx(b):
            pltpu.make_async_copy(g2_hbm.at[pl.ds(0, CHUNK)], idx_v.at[b],
                                  semi[b]).wait()

        def start_scatter(b):
            pltpu.async_copy(ones_v, acc_sh.at[idx_v.at[b]], sems_[b],
                             add=True)

        def wait_scatter(b):
            pltpu.make_async_copy(ones_v, acc_sh.at[idx_v.at[b]],
                                  sems_[b]).wait()

        def body(b, c):
            wait_idx(b)
            start_scatter(b)
            wait_scatter(b)
            if isinstance(c, int):
                if c + 2 < s_steps:
                    start_idx(b, c + 2)
            else:
                @pl.when(c + 2 < s_steps)
                def _():
                    start_idx(b, c + 2)

        start_idx(0, 0)
        start_idx(1, 1)
        body(0, 0)
        body(1, 1)

        @pl.loop(1, s_steps // 2)
        def _(kk):
            c0 = 2 * kk
            for b in range(2):
                body(b, c0 + b)

        for c in range(2 * (s_steps // 2), s_steps):
            body(c % 2, c)

        _fence()
        pltpu.sync_copy(acc_sh.at[pl.ds(sid * per_tile, per_tile)], buf_v)
        pltpu.sync_copy(buf_v, out_hbm.at[cid, pl.ds(sid * per_tile, per_tile)])

    return k(gidx2, zeros_hbm_arr, ones_hbm_arr)


def _tc_norm(counts_parts, rn, rnp):
    rows = rnp // 128
    live = rn // 128

    def body(c_ref, o_ref):
        c = c_ref[0] + c_ref[1]
        row = lax.broadcasted_iota(jnp.int32, (rows, 128), 0)
        o_ref[...] = jnp.where((row < live) & (c > 0.0), 1.0 / c, 0.0)

    out = pl.pallas_call(
        body,
        grid=(1,),
        in_specs=[pl.BlockSpec((NC, rows, 128), lambda i: (0, 0, 0))],
        out_specs=pl.BlockSpec((rows, 128), lambda i: (0, 0)),
        out_shape=jax.ShapeDtypeStruct((rows, 128), jnp.float32),
    )(counts_parts.reshape(NC, rows, 128))
    return out.reshape(rnp)


def _sc_message(hw, edata, ntab, zeros_hbm_arr, n, hd, s_steps):
    mesh = plsc.VectorSubcoreMesh(core_axis_name="c", subcore_axis_name="s")
    nfull = n // CHUNK
    ntail = n - nfull * CHUNK
    nzch = nfull + (1 if ntail else 0)

    @functools.partial(
        pl.kernel,
        out_type=jax.ShapeDtypeStruct((NC, n, hd), jnp.float32),
        mesh=mesh,
        scratch_types=[
            pltpu.VMEM_SHARED((n, hd), jnp.float32),
            pltpu.VMEM((3, 3, CHUNK), jnp.int32),
            pltpu.VMEM((3, CHUNK), jnp.float32),
            pltpu.VMEM((3, CHUNK, hd), jnp.float32),
        ] + [pltpu.SemaphoreType.DMA] * 12,
    )
    def k(hw_hbm, ed_hbm, ntab_hbm, z_hbm, out_hbm,
          acc_sh, idx_v, nrm_v, rows_v, *sems):
        cid = lax.axis_index("c")
        sid = lax.axis_index("s")
        wid = sid * NC + cid
        semi = sems[0:3]
        semr = sems[3:6]
        semn = sems[6:9]
        sems_ = sems[9:12]

        pltpu.sync_copy(z_hbm, rows_v.at[0])
        for kk in range(-(-nzch // NS)):
            c = sid + kk * NS

            @pl.when(c < nfull)
            def _():
                pltpu.sync_copy(rows_v.at[0], acc_sh.at[pl.ds(c * CHUNK, CHUNK)])

            if ntail:
                @pl.when(c == nfull)
                def _():
                    pltpu.sync_copy(rows_v.at[0, pl.ds(0, ntail)],
                                    acc_sh.at[pl.ds(nfull * CHUNK, ntail)])

        _fence()

        base = wid * s_steps

        def start_idx(b, c):
            pltpu.async_copy(ed_hbm.at[base + c], idx_v.at[b], semi[b])

        def wait_idx(b):
            pltpu.make_async_copy(ed_hbm.at[base], idx_v.at[b], semi[b]).wait()

        def start_gathers(b):
            pltpu.async_copy(hw_hbm.at[idx_v.at[b, 0]], rows_v.at[b], semr[b])
            pltpu.async_copy(ntab_hbm.at[idx_v.at[b, 2]], nrm_v.at[b], semn[b])

        def wait_gathers(b):
            pltpu.make_async_copy(
                hw_hbm.at[idx_v.at[b, 0]], rows_v.at[b], semr[b]).wait()
            pltpu.make_async_copy(
                ntab_hbm.at[idx_v.at[b, 2]], nrm_v.at[b], semn[b]).wait()

        def scale(b):
            @pl.loop(0, CHUNK, step=16)
            def _(j):
                nv = nrm_v[b, pl.ds(j, 16)]
                for jj in range(16):
                    sv = nv[jj]
                    for kk in range(hd // 16):
                        sl = (b, j + jj, pl.ds(kk * 16, 16))
                        rows_v.at[*sl][...] = rows_v.at[*sl][...] * sv

        def start_scatter(b):
            pltpu.async_copy(rows_v.at[b], acc_sh.at[idx_v.at[b, 1]], sems_[b],
                             add=True)

        def wait_scatter(b):
            pltpu.make_async_copy(
                rows_v.at[b], acc_sh.at[idx_v.at[b, 1]], sems_[b]).wait()

        def body(i, c, first=False):
            wait_gathers(i)
            scale(i)
            start_scatter(i)
            s2 = (i + 2) % 3
            if first:
                start_idx(s2, c + 2)
                wait_idx(s2)
                start_gathers(s2)
            elif isinstance(c, int):
                wait_scatter(s2)
                if c + 2 < s_steps:
                    start_idx(s2, c + 2)
                    wait_idx(s2)
                    start_gathers(s2)
            else:
                wait_scatter(s2)

                @pl.when(c + 2 < s_steps)
                def _():
                    start_idx(s2, c + 2)
                    wait_idx(s2)
                    start_gathers(s2)

        start_idx(0, 0)
        wait_idx(0)
        start_gathers(0)
        start_idx(1, 1)
        wait_idx(1)
        start_gathers(1)

        body(0, 0, first=True)
        for c in range(1, 3):
            body(c % 3, c)

        @pl.loop(1, s_steps // 3)
        def _(kk):
            c0 = 3 * kk
            for i in range(3):
                body(i, c0 + i)

        for c in range(3 * (s_steps // 3), s_steps):
            body(c % 3, c)
        wait_scatter((s_steps - 1) % 3)

        _fence()
        for kk in range(-(-nzch // NS)):
            c = sid + kk * NS

            @pl.when(c < nfull)
            def _():
                pltpu.sync_copy(acc_sh.at[pl.ds(c * CHUNK, CHUNK)], rows_v.at[0])
                pltpu.sync_copy(rows_v.at[0],
                                out_hbm.at[cid, pl.ds(c * CHUNK, CHUNK)])

            if ntail:
                @pl.when(c == nfull)
                def _():
                    pltpu.sync_copy(acc_sh.at[pl.ds(nfull * CHUNK, ntail)],
                                    rows_v.at[0, pl.ds(0, ntail)])
                    pltpu.sync_copy(rows_v.at[0, pl.ds(0, ntail)],
                                    out_hbm.at[cid, pl.ds(nfull * CHUNK, ntail)])

    return k(hw, edata, ntab, zeros_hbm_arr)


def kernel(x, edge_index, edge_type, W1, W1_self, W2, W2_self):
    n, d = x.shape
    r = W1.shape[0]
    hd = W1.shape[2]
    e = edge_index.shape[1]
    src = edge_index[0]
    dst = edge_index[1]
    et = edge_type.astype(jnp.int32)

    rn = r * n
    rnp = rn + (-rn % 2048) + 2048

    gidx = et * n + src
    gidx2 = et * n + dst

    s_steps = -(-e // (NW * CHUNK))
    e_pad = NW * CHUNK * s_steps
    pad = e_pad - e
    pad_i = jnp.arange(pad, dtype=jnp.int32)
    gidx_p = jnp.concatenate([gidx, pad_i % rn])
    dst_p = jnp.concatenate([dst, pad_i % n])
    gidx2_p = jnp.concatenate([gidx2, rn + pad_i % 1024])
    ctot = e_pad // CHUNK
    edata = jnp.stack(
        [gidx_p.reshape(ctot, CHUNK), dst_p.reshape(ctot, CHUNK),
         gidx2_p.reshape(ctot, CHUNK)], axis=1)

    w1a = jnp.concatenate([W1, W1_self[None]], axis=0)
    w2a = jnp.concatenate([W2, W2_self[None]], axis=0)

    z1d = jnp.zeros((rnp // NS,), jnp.float32)
    z2d = jnp.zeros((CHUNK, hd), jnp.float32)
    ones_c = jnp.ones((CHUNK,), jnp.float32)

    counts_parts = _sc_counts(gidx2_p, z1d, ones_c, rnp, s_steps)
    ntab = _tc_norm(counts_parts, rn, rnp)

    hw1 = _tc_matmul(x, w1a)
    parts1 = _sc_message(hw1, edata, ntab, z2d, n, hd, s_steps)
    hw2 = _tc_matmul_fused(parts1, hw1, w2a, n)
    parts2 = _sc_message(hw2, edata, ntab, z2d, n, hd, s_steps)
    return _tc_combine(parts2, hw2, n, relu=False)


# speedup vs baseline: 1.1123x; 1.0407x over previous
import functools

import jax
import jax.numpy as jnp
from jax import lax
from jax.experimental import pallas as pl
from jax.experimental.pallas import tpu as pltpu
from jax.experimental.pallas import tpu_sc as plsc

NC = 2
NS = 16
NW = NC * NS
CHUNK = 112


def _tc_matmul(h, w_all):
    n, d = h.shape
    rp, _, hd = w_all.shape
    bn = 1000

    def body(h_ref, w_ref, o_ref):
        o_ref[...] = jnp.dot(h_ref[...], w_ref[0],
                             preferred_element_type=jnp.float32)

    return pl.pallas_call(
        body,
        grid=(n // bn, rp),
        in_specs=[
            pl.BlockSpec((bn, d), lambda i, r: (i, 0)),
            pl.BlockSpec((1, d, hd), lambda i, r: (r, 0, 0)),
        ],
        out_specs=pl.BlockSpec((bn, hd), lambda i, r: (r * (n // bn) + i, 0)),
        out_shape=jax.ShapeDtypeStruct((rp * n, hd), jnp.float32),
    )(h, w_all)


def _tc_matmul_fused(parts, hw_prev, w_all, n):
    _, _, hd = parts.shape
    rp, d, _ = w_all.shape
    bn = 1000
    off = (hw_prev.shape[0] - n) // bn

    def body(p_ref, s_ref, w_ref, o_ref):
        h = jnp.maximum(p_ref[0] + p_ref[1] + s_ref[...], 0.0)
        o_ref[...] = jnp.dot(h, w_ref[0], preferred_element_type=jnp.float32)

    return pl.pallas_call(
        body,
        grid=(n // bn, rp),
        in_specs=[
            pl.BlockSpec((2, bn, hd), lambda i, r: (0, i, 0)),
            pl.BlockSpec((bn, hd), lambda i, r: (off + i, 0)),
            pl.BlockSpec((1, d, hd), lambda i, r: (r, 0, 0)),
        ],
        out_specs=pl.BlockSpec((bn, hd), lambda i, r: (r * (n // bn) + i, 0)),
        out_shape=jax.ShapeDtypeStruct((rp * n, hd), jnp.float32),
    )(parts, hw_prev, w_all)


def _tc_combine(parts, hw_full, n, relu):
    _, _, hd = parts.shape
    rn = hw_full.shape[0] - n
    bn = 1000
    off = rn // bn

    def body(p_ref, s_ref, o_ref):
        v = p_ref[0] + p_ref[1] + s_ref[...]
        o_ref[...] = jnp.maximum(v, 0.0) if relu else v

    return pl.pallas_call(
        body,
        grid=(n // bn,),
        in_specs=[
            pl.BlockSpec((2, bn, hd), lambda i: (0, i, 0)),
            pl.BlockSpec((bn, hd), lambda i: (off + i, 0)),
        ],
        out_specs=pl.BlockSpec((bn, hd), lambda i: (i, 0)),
        out_shape=jax.ShapeDtypeStruct((n, hd), jnp.float32),
    )(parts, hw_full)


def _fence():
    plsc.subcore_barrier()
    pl.delay(3000)
    plsc.subcore_barrier()


def _sc_counts(gidx2, zeros_hbm_arr, ones_hbm_arr, rnp, s_steps):
    mesh = plsc.VectorSubcoreMesh(core_axis_name="c", subcore_axis_name="s")
    per_tile = rnp // NS

    @functools.partial(
        pl.kernel,
        out_type=jax.ShapeDtypeStruct((NC, rnp), jnp.float32),
        mesh=mesh,
        scratch_types=[
            pltpu.VMEM_SHARED((rnp,), jnp.float32),
            pltpu.VMEM((2, CHUNK), jnp.int32),
            pltpu.VMEM((CHUNK,), jnp.float32),
            pltpu.VMEM((per_tile,), jnp.float32),
        ] + [pltpu.SemaphoreType.DMA] * 4,
    )
    def k(g2_hbm, z_hbm, ones_hbm, out_hbm, acc_sh, idx_v, ones_v, buf_v,
          semi0, semi1, sems0, sems1):
        cid = lax.axis_index("c")
        sid = lax.axis_index("s")
        wid = sid * NC + cid
        semi = (semi0, semi1)
        sems_ = (sems0, sems1)

        pltpu.sync_copy(ones_hbm, ones_v)
        pltpu.sync_copy(z_hbm.at[pl.ds(0, per_tile)], buf_v)
        pltpu.sync_copy(buf_v, acc_sh.at[pl.ds(sid * per_tile, per_tile)])
        _fence()

        base = wid * s_steps

        def start_idx(b, c):
            pltpu.async_copy(g2_hbm.at[pl.ds((base + c) * CHUNK, CHUNK)],
                             idx_v.at[b], semi[b])

        def wait_idx(b):
            pltpu.make_async_copy(g2_hbm.at[pl.ds(0, CHUNK)], idx_v.at[b],
                                  semi[b]).wait()

        def start_scatter(b):
            pltpu.async_copy(ones_v, acc_sh.at[idx_v.at[b]], sems_[b],
                             add=True)

        def wait_scatter(b):
            pltpu.make_async_copy(ones_v, acc_sh.at[idx_v.at[b]],
                                  sems_[b]).wait()

        def body(b, c):
            wait_idx(b)
            start_scatter(b)
            wait_scatter(b)
            if isinstance(c, int):
                if c + 2 < s_steps:
                    start_idx(b, c + 2)
            else:
                @pl.when(c + 2 < s_steps)
                def _():
                    start_idx(b, c + 2)

        start_idx(0, 0)
        start_idx(1, 1)
        body(0, 0)
        body(1, 1)

        @pl.loop(1, s_steps // 2)
        def _(kk):
            c0 = 2 * kk
            for b in range(2):
                body(b, c0 + b)

        for c in range(2 * (s_steps // 2), s_steps):
            body(c % 2, c)

        _fence()
        pltpu.sync_copy(acc_sh.at[pl.ds(sid * per_tile, per_tile)], buf_v)
        pltpu.sync_copy(buf_v, out_hbm.at[cid, pl.ds(sid * per_tile, per_tile)])

    return k(gidx2, zeros_hbm_arr, ones_hbm_arr)


def _tc_norm(counts_parts, rn, rnp):
    rows = rnp // 128
    live = rn // 128

    def body(c_ref, o_ref):
        c = c_ref[0] + c_ref[1]
        row = lax.broadcasted_iota(jnp.int32, (rows, 128), 0)
        o_ref[...] = jnp.where((row < live) & (c > 0.0), 1.0 / c, 0.0)

    out = pl.pallas_call(
        body,
        grid=(1,),
        in_specs=[pl.BlockSpec((NC, rows, 128), lambda i: (0, 0, 0))],
        out_specs=pl.BlockSpec((rows, 128), lambda i: (0, 0)),
        out_shape=jax.ShapeDtypeStruct((rows, 128), jnp.float32),
    )(counts_parts.reshape(NC, rows, 128))
    return out.reshape(rnp)


def _sc_message(hw, edata, ntab, zeros_hbm_arr, n, hd, s_steps):
    mesh = plsc.VectorSubcoreMesh(core_axis_name="c", subcore_axis_name="s")
    nfull = n // CHUNK
    ntail = n - nfull * CHUNK
    nzch = nfull + (1 if ntail else 0)

    @functools.partial(
        pl.kernel,
        out_type=jax.ShapeDtypeStruct((NC, n, hd), jnp.float32),
        mesh=mesh,
        scratch_types=[
            pltpu.VMEM_SHARED((n, hd), jnp.float32),
            pltpu.VMEM((6, 3, CHUNK), jnp.int32),
            pltpu.VMEM((3, CHUNK), jnp.float32),
            pltpu.VMEM((3, CHUNK, hd), jnp.float32),
        ] + [pltpu.SemaphoreType.DMA] * 15,
    )
    def k(hw_hbm, ed_hbm, ntab_hbm, z_hbm, out_hbm,
          acc_sh, idx_v, nrm_v, rows_v, *sems):
        cid = lax.axis_index("c")
        sid = lax.axis_index("s")
        wid = sid * NC + cid
        semi = sems[0:6]
        semr = sems[6:9]
        semn = sems[9:12]
        sems_ = sems[12:15]

        pltpu.sync_copy(z_hbm, rows_v.at[0])
        for kk in range(-(-nzch // NS)):
            c = sid + kk * NS

            @pl.when(c < nfull)
            def _():
                pltpu.sync_copy(rows_v.at[0], acc_sh.at[pl.ds(c * CHUNK, CHUNK)])

            if ntail:
                @pl.when(c == nfull)
                def _():
                    pltpu.sync_copy(rows_v.at[0, pl.ds(0, ntail)],
                                    acc_sh.at[pl.ds(nfull * CHUNK, ntail)])

        _fence()

        base = wid * s_steps

        def start_idx(b, c):
            pltpu.async_copy(ed_hbm.at[base + c], idx_v.at[b], semi[b])

        def wait_idx(b):
            pltpu.make_async_copy(ed_hbm.at[base], idx_v.at[b], semi[b]).wait()

        def start_gathers(b, j):
            pltpu.async_copy(hw_hbm.at[idx_v.at[j, 0]], rows_v.at[b], semr[b])
            pltpu.async_copy(ntab_hbm.at[idx_v.at[j, 2]], nrm_v.at[b], semn[b])

        def wait_gathers(b, j):
            pltpu.make_async_copy(
                hw_hbm.at[idx_v.at[j, 0]], rows_v.at[b], semr[b]).wait()
            pltpu.make_async_copy(
                ntab_hbm.at[idx_v.at[j, 2]], nrm_v.at[b], semn[b]).wait()

        def scale(b):
            @pl.loop(0, CHUNK, step=16)
            def _(j):
                nv = nrm_v[b, pl.ds(j, 16)]
                for jj in range(16):
                    sv = nv[jj]
                    for kk in range(hd // 16):
                        sl = (b, j + jj, pl.ds(kk * 16, 16))
                        rows_v.at[*sl][...] = rows_v.at[*sl][...] * sv

        def start_scatter(b, j):
            pltpu.async_copy(rows_v.at[b], acc_sh.at[idx_v.at[j, 1]], sems_[b],
                             add=True)

        def wait_scatter(b):
            pltpu.make_async_copy(
                rows_v.at[b], acc_sh.at[idx_v.at[0, 1]], sems_[b]).wait()

        def body(i, j, c, first=False):
            wait_gathers(i, j)
            scale(i)
            start_scatter(i, j)
            if not first:
                wait_scatter((i + 2) % 3)

            def prep():
                start_idx((j + 3) % 6, c + 3)

            def launch():
                wait_idx((j + 2) % 6)
                start_gathers((i + 2) % 3, (j + 2) % 6)

            if isinstance(c, int):
                if c + 3 < s_steps:
                    prep()
                if c + 2 < s_steps:
                    launch()
            else:
                @pl.when(c + 3 < s_steps)
                def _():
                    prep()

                @pl.when(c + 2 < s_steps)
                def _():
                    launch()

        start_idx(0, 0)
        start_idx(1, 1)
        start_idx(2, 2)
        wait_idx(0)
        start_gathers(0, 0)
        wait_idx(1)
        start_gathers(1, 1)

        body(0, 0, 0, first=True)
        for c in range(1, 6):
            body(c % 3, c % 6, c)

        @pl.loop(1, s_steps // 6)
        def _(kk):
            c0 = 6 * kk
            for u in range(6):
                body(u % 3, u, c0 + u)

        for c in range(6 * (s_steps // 6), s_steps):
            body(c % 3, c % 6, c)
        wait_scatter((s_steps - 1) % 3)

        _fence()
        for kk in range(-(-nzch // NS)):
            c = sid + kk * NS

            @pl.when(c < nfull)
            def _():
                pltpu.sync_copy(acc_sh.at[pl.ds(c * CHUNK, CHUNK)], rows_v.at[0])
                pltpu.sync_copy(rows_v.at[0],
                                out_hbm.at[cid, pl.ds(c * CHUNK, CHUNK)])

            if ntail:
                @pl.when(c == nfull)
                def _():
                    pltpu.sync_copy(acc_sh.at[pl.ds(nfull * CHUNK, ntail)],
                                    rows_v.at[0, pl.ds(0, ntail)])
                    pltpu.sync_copy(rows_v.at[0, pl.ds(0, ntail)],
                                    out_hbm.at[cid, pl.ds(nfull * CHUNK, ntail)])

    return k(hw, edata, ntab, zeros_hbm_arr)


def kernel(x, edge_index, edge_type, W1, W1_self, W2, W2_self):
    n, d = x.shape
    r = W1.shape[0]
    hd = W1.shape[2]
    e = edge_index.shape[1]
    src = edge_index[0]
    dst = edge_index[1]
    et = edge_type.astype(jnp.int32)

    rn = r * n
    rnp = rn + (-rn % 2048) + 2048

    gidx = et * n + src
    gidx2 = et * n + dst

    s_steps = -(-e // (NW * CHUNK))
    e_pad = NW * CHUNK * s_steps
    pad = e_pad - e
    pad_i = jnp.arange(pad, dtype=jnp.int32)
    gidx_p = jnp.concatenate([gidx, pad_i % rn])
    dst_p = jnp.concatenate([dst, pad_i % n])
    gidx2_p = jnp.concatenate([gidx2, rn + pad_i % 1024])
    ctot = e_pad // CHUNK
    edata = jnp.stack(
        [gidx_p.reshape(ctot, CHUNK), dst_p.reshape(ctot, CHUNK),
         gidx2_p.reshape(ctot, CHUNK)], axis=1)

    w1a = jnp.concatenate([W1, W1_self[None]], axis=0)
    w2a = jnp.concatenate([W2, W2_self[None]], axis=0)

    z1d = jnp.zeros((rnp // NS,), jnp.float32)
    z2d = jnp.zeros((CHUNK, hd), jnp.float32)
    ones_c = jnp.ones((CHUNK,), jnp.float32)

    counts_parts = _sc_counts(gidx2_p, z1d, ones_c, rnp, s_steps)
    ntab = _tc_norm(counts_parts, rn, rnp)

    hw1 = _tc_matmul(x, w1a)
    parts1 = _sc_message(hw1, edata, ntab, z2d, n, hd, s_steps)
    hw2 = _tc_matmul_fused(parts1, hw1, w2a, n)
    parts2 = _sc_message(hw2, edata, ntab, z2d, n, hd, s_steps)
    return _tc_combine(parts2, hw2, n, relu=False)
